# Initial kernel scaffold; baseline (speedup 1.0000x reference)
import jax
import jax.numpy as jnp
from jax.experimental import pallas as pl


def kernel(edge_features, triangle_features, edge_index, triangle_index, W_msg1, b_msg1, W_msg2, b_msg2, W_upd1, b_upd1, W_upd2, b_upd2, ln_g, ln_b):
    raise NotImplementedError("write your pallas kernel here")


# speedup vs baseline: 1.0102x; 1.0102x over previous
import functools

import jax
import jax.numpy as jnp
from jax.experimental import pallas as pl
from jax.experimental.pallas import tpu as pltpu


def _msg_mlp_body(x_ref, w1_ref, b1_ref, w2_ref, b2_ref, o_ref):
    x = x_ref[...]
    h = jnp.dot(x, w1_ref[...], preferred_element_type=jnp.float32) + b1_ref[...]
    h = h * jax.nn.sigmoid(h)
    y = jnp.dot(h, w2_ref[...], preferred_element_type=jnp.float32) + b2_ref[...]
    o_ref[...] = y * jax.nn.sigmoid(y)


def _msg_mlp(tri_scalars, W1, b1, W2, b2, blk):
    t, d = tri_scalars.shape
    hid = W1.shape[1]
    grid = t // blk
    return pl.pallas_call(
        _msg_mlp_body,
        grid=(grid,),
        in_specs=[
            pl.BlockSpec((blk, d), lambda i: (i, 0)),
            pl.BlockSpec((d, hid), lambda i: (0, 0)),
            pl.BlockSpec((1, hid), lambda i: (0, 0)),
            pl.BlockSpec((hid, hid), lambda i: (0, 0)),
            pl.BlockSpec((1, hid), lambda i: (0, 0)),
        ],
        out_specs=pl.BlockSpec((blk, hid), lambda i: (i, 0)),
        out_shape=jax.ShapeDtypeStruct((t, hid), jnp.float32),
    )(tri_scalars, W1, b1.reshape(1, hid), W2, b2.reshape(1, hid))


def _upd_body(es_ref, agg_ref, cnt_ref, w1a_ref, w1b_ref, b1_ref, w2_ref,
              b2_ref, g_ref, bb_ref, o_ref):
    es = es_ref[...]
    cnt = cnt_ref[...]
    mean_msg = agg_ref[...] / jnp.maximum(cnt, 1.0)
    u = (jnp.dot(es, w1a_ref[...], preferred_element_type=jnp.float32)
         + jnp.dot(mean_msg, w1b_ref[...], preferred_element_type=jnp.float32)
         + b1_ref[...])
    u = u * jax.nn.sigmoid(u)
    upd = jnp.dot(u, w2_ref[...], preferred_element_type=jnp.float32) + b2_ref[...]
    r = es + upd
    m = jnp.mean(r, axis=-1, keepdims=True)
    v = jnp.mean((r - m) * (r - m), axis=-1, keepdims=True)
    o_ref[...] = (r - m) * jax.lax.rsqrt(v + 1e-5) * g_ref[...] + bb_ref[...]


def _upd_mlp(edge_scalars, agg, cnt, W1, b1, W2, b2, g, b, blk):
    e, d = edge_scalars.shape
    hid = W1.shape[1]
    W1a = W1[:d]
    W1b = W1[d:]
    grid = e // blk
    return pl.pallas_call(
        _upd_body,
        grid=(grid,),
        in_specs=[
            pl.BlockSpec((blk, d), lambda i: (i, 0)),
            pl.BlockSpec((blk, hid), lambda i: (i, 0)),
            pl.BlockSpec((blk, 1), lambda i: (i, 0)),
            pl.BlockSpec((d, hid), lambda i: (0, 0)),
            pl.BlockSpec((hid, hid), lambda i: (0, 0)),
            pl.BlockSpec((1, hid), lambda i: (0, 0)),
            pl.BlockSpec((hid, d), lambda i: (0, 0)),
            pl.BlockSpec((1, d), lambda i: (0, 0)),
            pl.BlockSpec((1, d), lambda i: (0, 0)),
            pl.BlockSpec((1, d), lambda i: (0, 0)),
        ],
        out_specs=pl.BlockSpec((blk, d), lambda i: (i, 0)),
        out_shape=jax.ShapeDtypeStruct((e, d), jnp.float32),
    )(edge_scalars, agg, cnt.reshape(e, 1), W1a, W1b, b1.reshape(1, hid),
      W2, b2.reshape(1, d), g.reshape(1, d), b.reshape(1, d))


def kernel(edge_features, triangle_features, edge_index, triangle_index,
           W_msg1, b_msg1, W_msg2, b_msg2, W_upd1, b_upd1, W_upd2, b_upd2,
           ln_g, ln_b):
    num_edges = edge_features.shape[0]
    num_tri = triangle_features.shape[0]
    hid = W_msg1.shape[1]

    M = jnp.maximum(jnp.max(edge_index), jnp.max(triangle_index)) + 1
    src, tgt = edge_index[0], edge_index[1]
    edge_key = jnp.minimum(src, tgt) * M + jnp.maximum(src, tgt)
    order = jnp.argsort(edge_key).astype(jnp.int32)
    sorted_keys = edge_key[order]

    v0, v1, v2 = triangle_index[0], triangle_index[1], triangle_index[2]

    def pkey(x, y):
        return jnp.minimum(x, y) * M + jnp.maximum(x, y)

    tri_keys = jnp.concatenate([pkey(v0, v1), pkey(v1, v2), pkey(v2, v0)])
    pos = jnp.searchsorted(sorted_keys, tri_keys, side='right') - 1
    posc = jnp.clip(pos, 0, num_edges - 1)
    valid = (pos >= 0) & (sorted_keys[posc] == tri_keys)
    match_edge = jnp.where(valid, order[posc], num_edges)

    blk_t = 1000 if num_tri % 1000 == 0 else num_tri
    tri_msgs = _msg_mlp(triangle_features[:, 3:], W_msg1, b_msg1, W_msg2,
                        b_msg2, blk_t)

    tri_ids = jnp.tile(jnp.arange(num_tri, dtype=jnp.int32), 3)
    contrib = tri_msgs[tri_ids]
    agg = jnp.zeros((num_edges + 1, hid), jnp.float32).at[match_edge].add(contrib)
    cnt = jnp.zeros((num_edges + 1,), jnp.float32).at[match_edge].add(1.0)
    agg = agg[:num_edges]
    cnt = cnt[:num_edges]

    blk_e = 1280 if num_edges % 1280 == 0 else num_edges
    res = _upd_mlp(edge_features[:, 3:], agg, cnt, W_upd1, b_upd1, W_upd2,
                   b_upd2, ln_g, ln_b, blk_e)
    return jnp.concatenate([edge_features[:, :3], res], axis=-1)
